# baseline (speedup 1.0000x reference)
import jax
import jax.numpy as jnp
from jax.experimental import pallas as pl
from jax.experimental.pallas import tpu as pltpu

_XR = 2048
_ZR = 512


def _x_body(y_ref, x_ref, o_ref):
    i = pl.program_id(0)

    @pl.when(i == 0)
    def _patch():
        blk = x_ref[...]
        r = jax.lax.broadcasted_iota(jnp.int32, (_XR, 256), 0)
        b = jnp.where(r == 10, y_ref[0, :][None, :], blk)
        b = jnp.where(r == 2, y_ref[1, :][None, :], b)
        b = jnp.where(r == 1, jnp.float32(45.0), b)
        o_ref[...] = b

    @pl.when(i != 0)
    def _copy():
        o_ref[...] = x_ref[...]


def _z_body(w_ref, z_ref, o_ref):
    i = pl.program_id(0)

    @pl.when(i == 0)
    def _patch():
        blk = z_ref[...]
        r = jax.lax.broadcasted_iota(jnp.int32, (_ZR, 1024), 0)
        c = jax.lax.broadcasted_iota(jnp.int32, (_ZR, 1024), 1)
        add = (w_ref[0] * ((r == 1) & (c == 3)).astype(jnp.float32)
               + w_ref[1] * ((r == 0) & (c == 2)).astype(jnp.float32)
               + w_ref[2] * ((r == 0) & (c == 1)).astype(jnp.float32))
        o_ref[...] = blk + add

    @pl.when(i != 0)
    def _copy():
        o_ref[...] = z_ref[...]


def kernel(x, y, z, w):
    xo = pl.pallas_call(
        _x_body,
        grid=(x.shape[0] // _XR,),
        in_specs=[
            pl.BlockSpec((2, 256), lambda i: (0, 0)),
            pl.BlockSpec((_XR, 256), lambda i: (i, 0)),
        ],
        out_specs=pl.BlockSpec((_XR, 256), lambda i: (i, 0)),
        out_shape=jax.ShapeDtypeStruct(x.shape, x.dtype),
    )(y, x)
    zo = pl.pallas_call(
        _z_body,
        grid=(z.shape[0] // _ZR,),
        in_specs=[
            pl.BlockSpec(memory_space=pltpu.SMEM),
            pl.BlockSpec((_ZR, 1024), lambda i: (i, 0)),
        ],
        out_specs=pl.BlockSpec((_ZR, 1024), lambda i: (i, 0)),
        out_shape=jax.ShapeDtypeStruct(z.shape, z.dtype),
    )(w, z)
    return (xo, zo)
